# baseline (speedup 1.0000x reference)
import functools

import jax
import jax.numpy as jnp
import numpy as np
from jax import lax
from jax.experimental import pallas as pl
from jax.experimental.pallas import tpu as pltpu
from jax.experimental.pallas import tpu_sc as plsc

BINS_N = 10
MU_C = 0.02
MU2_C = MU_C * MU_C
MMT_C = 0.7
ALPHA_C = 0.25
EPS_C = 1e-5

B_N, H_N, W_N, C_N = 64, 256, 256, 5
N_PLANES = 3 + 2 * (BINS_N - 1)


def _edge_list():
    e = [float(x) / BINS_N for x in range(BINS_N + 1)]
    e[-1] = 1000.0
    return [np.float32(v) for v in e]


STRIP = 16
N_STRIPS = H_N // STRIP


def _plane_kernel(p_ref, t_ref, out_ref):
    i = pl.program_id(0)
    edges = _edge_list()

    def fold(x):
        a = x[:, :128] + x[:, 128:]
        return a[:8, :] + a[8:, :]

    def one_strip(sl, accs):
        pc = p_ref[0, sl, :]
        tc = t_ref[0, sl, :]

        u = 2.0 * tc - 1.0
        one_m_t = 1.0 - tc
        x_t = pc * u + one_m_t
        alpha_t = ALPHA_C * u + one_m_t
        om = 1.0 - x_t
        fl = -alpha_t * om * om * jnp.log(x_t + EPS_C)

        v = jnp.where(tc > 0.1, 1.0, 0.0)

        new = list(accs)
        new[0] = new[0] + fold(fl)
        new[1] = new[1] + fold(v)

        for c in range(1, C_N):
            dpc = p_ref[c, sl, :] - t_ref[c, sl, :]
            s = dpc * dpc + MU2_C
            rs = jax.lax.rsqrt(s)
            lc = s * rs - MU_C
            vl = v * lc
            gv = jnp.abs(dpc) * rs * v
            new[2] = new[2] + fold(vl)
            q = 3
            for b in range(1, BINS_N):
                m = gv >= edges[b]
                sv = jnp.where(m, 1.0, 0.0)
                wv = jnp.where(m, vl, 0.0)
                new[q] = new[q] + fold(sv)
                new[q + 1] = new[q + 1] + fold(wv)
                q += 2
        return tuple(new)

    def strip_body(j, accs):
        for u in range(4):
            accs = one_strip(pl.ds((j * 4 + u) * STRIP, STRIP), accs)
        return accs

    zero = jnp.zeros((8, 128), jnp.float32)
    accs = jax.lax.fori_loop(
        0, N_STRIPS // 4, strip_body, tuple(zero for _ in range(N_PLANES)))
    vals = jnp.stack(accs)

    @pl.when(i == 0)
    def _():
        out_ref[...] = vals

    @pl.when(i > 0)
    def _():
        out_ref[...] += vals


def _streaming_pass(p3d, t3d):
    return pl.pallas_call(
        _plane_kernel,
        grid=(B_N - K_SC,),
        in_specs=[
            pl.BlockSpec((C_N, H_N, W_N), lambda i: (i + K_SC, 0, 0)),
            pl.BlockSpec((C_N, H_N, W_N), lambda i: (i + K_SC, 0, 0)),
        ],
        out_specs=pl.BlockSpec((N_PLANES, 8, 128), lambda i: (0, 0, 0)),
        out_shape=jax.ShapeDtypeStruct((N_PLANES, 8, 128), jnp.float32),
        compiler_params=pltpu.CompilerParams(
            dimension_semantics=("arbitrary",),
        ),
    )(p3d, t3d)


K_SC = 8
NC, NS, LANES = 2, 16, 16
NW = NC * NS
TASKS = K_SC * 4 * 16
TASKS_PER_W = TASKS // NW
SC_ROW = 384
RSQRT_MAGIC = np.int32(0x5F3759DF)


def _sc_body(p_hbm, t_hbm, out_hbm, pbuf, tbuf, cbuf, stage):
    wid = lax.axis_index("s") * NC + lax.axis_index("c")
    lane = lax.iota(jnp.int32, LANES)
    zero16 = jnp.zeros((LANES,), jnp.float32)
    for i in range(SC_ROW // LANES):
        stage[pl.ds(i * LANES, LANES)] = zero16

    def task_body(ti, carry):
        t = wid + ti * NW
        b = t >> 6
        rest = t & 63
        ch = (rest >> 4) + 1
        rb = rest & 15
        rs0 = rb * 16
        pltpu.sync_copy(p_hbm.at[b * C_N + ch, pl.ds(rs0, 16)], pbuf)
        pltpu.sync_copy(t_hbm.at[b * C_N + ch, pl.ds(rs0, 16)], tbuf)
        pltpu.sync_copy(t_hbm.at[b * C_N, pl.ds(rs0, 16)], cbuf)

        def group(g, c2):
            r = g >> 4
            col = (g & 15) << 4
            tc0 = cbuf[r, pl.ds(col, LANES)]
            vf = jnp.where(tc0 > 0.1, 1.0, 0.0).astype(jnp.float32)
            pc = pbuf[r, pl.ds(col, LANES)]
            tcc = tbuf[r, pl.ds(col, LANES)]
            d = pc - tcc
            s = d * d + MU2_C
            bits = plsc.bitcast(s, jnp.int32)
            y = plsc.bitcast(RSQRT_MAGIC - (bits >> 1), jnp.float32)
            y = y * (1.5 - 0.5 * s * y * y)
            y = y * (1.5 - 0.5 * s * y * y)
            y = y * (1.5 - 0.5 * s * y * y)
            lc = s * y - MU_C
            vl = vf * lc
            gv = jnp.abs(d) * y * vf
            k = jnp.minimum((gv * 10.0).astype(jnp.int32), 9)
            addr = lax.shift_left(k, 4) + lane
            plsc.addupdate_scatter(stage, [addr], vf)
            plsc.addupdate_scatter(stage, [addr + 160], vl)
            return c2

        return lax.fori_loop(0, 256, group, carry)

    lax.fori_loop(0, TASKS_PER_W, task_body, jnp.int32(0))
    pltpu.sync_copy(stage, out_hbm.at[pl.ds(wid * SC_ROW, SC_ROW)])


def _sc_pass(p3d, t3d):
    mesh = plsc.VectorSubcoreMesh(
        core_axis_name="c", subcore_axis_name="s",
        num_cores=NC, num_subcores=NS,
    )
    f = functools.partial(
        pl.kernel,
        out_type=jax.ShapeDtypeStruct((NW * SC_ROW,), jnp.float32),
        mesh=mesh,
        scratch_types=[
            pltpu.VMEM((16, W_N), jnp.float32),
            pltpu.VMEM((16, W_N), jnp.float32),
            pltpu.VMEM((16, W_N), jnp.float32),
            pltpu.VMEM((SC_ROW,), jnp.float32),
        ],
        compiler_params=pltpu.CompilerParams(
            needs_layout_passes=False,
        ),
    )(_sc_body)
    return f(p3d, t3d)


def _focal_mini_kernel(p_ref, t_ref, out_ref):
    i = pl.program_id(0)
    pc = p_ref[0]
    tc = t_ref[0]
    u = 2.0 * tc - 1.0
    one_m_t = 1.0 - tc
    x_t = pc * u + one_m_t
    alpha_t = ALPHA_C * u + one_m_t
    om = 1.0 - x_t
    fl = -alpha_t * om * om * jnp.log(x_t + EPS_C)
    v = jnp.where(tc > 0.1, 1.0, 0.0)

    def red(x):
        a = x[:, :128] + x[:, 128:]
        b = a.reshape(H_N // 8, 8, 128)
        return jnp.sum(b, axis=0)

    vals = jnp.stack([red(fl), red(v)])

    @pl.when(i == 0)
    def _():
        out_ref[...] = vals

    @pl.when(i > 0)
    def _():
        out_ref[...] += vals


def _focal_mini(p3d, t3d):
    return pl.pallas_call(
        _focal_mini_kernel,
        grid=(K_SC,),
        in_specs=[
            pl.BlockSpec((1, H_N, W_N), lambda i: (i * C_N, 0, 0)),
            pl.BlockSpec((1, H_N, W_N), lambda i: (i * C_N, 0, 0)),
        ],
        out_specs=pl.BlockSpec((2, 8, 128), lambda i: (0, 0, 0)),
        out_shape=jax.ShapeDtypeStruct((2, 8, 128), jnp.float32),
        compiler_params=pltpu.CompilerParams(
            dimension_semantics=("arbitrary",),
        ),
    )(p3d, t3d)


@jax.jit
def kernel(preds, targets):
    p3d = jnp.transpose(preds, (0, 3, 1, 2)).reshape(B_N * C_N, H_N, W_N)
    t3d = jnp.transpose(targets, (0, 3, 1, 2)).reshape(B_N * C_N, H_N, W_N)
    sc_rows = _sc_pass(p3d, t3d).reshape(NW, SC_ROW)
    sums = _streaming_pass(p3d, t3d).sum(axis=(1, 2))
    mini = _focal_mini(p3d, t3d).sum(axis=(1, 2))

    focal_sum = sums[0] + mini[0]
    tot_raw = sums[1] + mini[1]
    L0 = sums[2]
    S_rest = sums[3::2]
    L_rest = sums[4::2]

    counts_sc = sc_rows[:, 0:160].sum(axis=0).reshape(BINS_N, LANES).sum(axis=1)
    lsum_sc = sc_rows[:, 160:320].sum(axis=0).reshape(BINS_N, LANES).sum(axis=1)

    tot = jnp.maximum(tot_raw, 1.0)
    S = jnp.concatenate([jnp.reshape(4.0 * sums[1], (1,)), S_rest])
    L = jnp.concatenate([jnp.reshape(L0, (1,)), L_rest])
    counts = S - jnp.concatenate([S[1:], jnp.zeros((1,), jnp.float32)])
    lsum = L - jnp.concatenate([L[1:], jnp.zeros((1,), jnp.float32)])
    counts = counts + counts_sc
    lsum = lsum + lsum_sc

    acc_sum = (1.0 - MMT_C) * counts
    n = (counts > 0).astype(jnp.float32).sum()
    per_bin_w = jnp.where(counts > 0, tot / jnp.maximum(acc_sum, 1e-12), 0.0)
    reg = (lsum * per_bin_w).sum()
    reg = jnp.where(n > 0, reg / jnp.maximum(n, 1.0), reg)
    reg_loss = reg / tot

    cls_loss = focal_sum / (B_N * H_N * W_N)
    total = cls_loss + reg_loss
    return (total,
            jax.lax.stop_gradient(reg_loss),
            jax.lax.stop_gradient(cls_loss))


# speedup vs baseline: 1.3941x; 1.3941x over previous
import jax
import jax.numpy as jnp
import numpy as np
from jax.experimental import pallas as pl
from jax.experimental.pallas import tpu as pltpu

BINS_N = 10
MU_C = 0.02
MU2_C = MU_C * MU_C
MMT_C = 0.7
ALPHA_C = 0.25
EPS_C = 1e-5

B_N, H_N, W_N, C_N = 64, 256, 256, 5
N_PLANES = 3 + 2 * (BINS_N - 1)


def _edge_list():
    e = [float(x) / BINS_N for x in range(BINS_N + 1)]
    e[-1] = 1000.0
    return [np.float32(v) for v in e]


STRIP = 16
N_STRIPS = H_N // STRIP


def _plane_kernel(p_ref, t_ref, out_ref):
    i = pl.program_id(0)
    edges = _edge_list()

    def fold(x):
        a = x[:, :128] + x[:, 128:]
        return a[:8, :] + a[8:, :]

    def one_strip(sl, accs):
        pc = p_ref[0, sl, :]
        tc = t_ref[0, sl, :]

        u = 2.0 * tc - 1.0
        one_m_t = 1.0 - tc
        x_t = pc * u + one_m_t
        alpha_t = ALPHA_C * u + one_m_t
        om = 1.0 - x_t
        fl = -alpha_t * om * om * jnp.log(x_t + EPS_C)

        v = jnp.where(tc > 0.1, 1.0, 0.0)

        new = list(accs)
        new[0] = new[0] + fold(fl)
        new[1] = new[1] + fold(v)

        for c in range(1, C_N):
            dpc = p_ref[c, sl, :] - t_ref[c, sl, :]
            s = dpc * dpc + MU2_C
            rs = jax.lax.rsqrt(s)
            lc = s * rs - MU_C
            vl = v * lc
            gv = jnp.abs(dpc) * rs * v
            new[2] = new[2] + fold(vl)
            q = 3
            for b in range(1, BINS_N):
                m = gv >= edges[b]
                sv = jnp.where(m, 1.0, 0.0)
                wv = jnp.where(m, vl, 0.0)
                new[q] = new[q] + fold(sv)
                new[q + 1] = new[q + 1] + fold(wv)
                q += 2
        return tuple(new)

    def strip_body(j, accs):
        for u in range(4):
            accs = one_strip(pl.ds((j * 4 + u) * STRIP, STRIP), accs)
        return accs

    zero = jnp.zeros((8, 128), jnp.float32)
    accs = jax.lax.fori_loop(
        0, N_STRIPS // 4, strip_body, tuple(zero for _ in range(N_PLANES)))
    vals = jnp.stack(accs)

    @pl.when(i == 0)
    def _():
        out_ref[...] = vals

    @pl.when(i > 0)
    def _():
        out_ref[...] += vals


def _streaming_pass(p3d, t3d):
    return pl.pallas_call(
        _plane_kernel,
        grid=(B_N,),
        in_specs=[
            pl.BlockSpec((C_N, H_N, W_N), lambda i: (i, 0, 0)),
            pl.BlockSpec((C_N, H_N, W_N), lambda i: (i, 0, 0)),
        ],
        out_specs=pl.BlockSpec((N_PLANES, 8, 128), lambda i: (0, 0, 0)),
        out_shape=jax.ShapeDtypeStruct((N_PLANES, 8, 128), jnp.float32),
        compiler_params=pltpu.CompilerParams(
            dimension_semantics=("arbitrary",),
        ),
    )(p3d, t3d)


@jax.jit
def kernel(preds, targets):
    p3d = jnp.transpose(preds, (0, 3, 1, 2)).reshape(B_N * C_N, H_N, W_N)
    t3d = jnp.transpose(targets, (0, 3, 1, 2)).reshape(B_N * C_N, H_N, W_N)
    sums = _streaming_pass(p3d, t3d).sum(axis=(1, 2))

    focal_sum = sums[0]
    tot_raw = sums[1]
    L0 = sums[2]
    S_rest = sums[3::2]
    L_rest = sums[4::2]

    tot = jnp.maximum(tot_raw, 1.0)
    S = jnp.concatenate([jnp.reshape(4.0 * tot_raw, (1,)), S_rest])
    L = jnp.concatenate([jnp.reshape(L0, (1,)), L_rest])
    counts = S - jnp.concatenate([S[1:], jnp.zeros((1,), jnp.float32)])
    lsum = L - jnp.concatenate([L[1:], jnp.zeros((1,), jnp.float32)])

    acc_sum = (1.0 - MMT_C) * counts
    n = (counts > 0).astype(jnp.float32).sum()
    per_bin_w = jnp.where(counts > 0, tot / jnp.maximum(acc_sum, 1e-12), 0.0)
    reg = (lsum * per_bin_w).sum()
    reg = jnp.where(n > 0, reg / jnp.maximum(n, 1.0), reg)
    reg_loss = reg / tot

    cls_loss = focal_sum / (B_N * H_N * W_N)
    total = cls_loss + reg_loss
    return (total,
            jax.lax.stop_gradient(reg_loss),
            jax.lax.stop_gradient(cls_loss))
